# baseline (speedup 1.0000x reference)
import functools

import jax
import jax.numpy as jnp
from jax import lax
from jax.experimental import pallas as pl
from jax.experimental.pallas import tpu as pltpu
from jax.experimental.pallas import tpu_sc as plsc

EMBED_DIM = 64
NUM_WORKERS = 32
CHUNK = 512


def _emb_body(idx_hbm, table_hbm, out_hbm, idx_v, rows_v, sem, *, per_w, n_chunk):
    wid = lax.axis_index("s") * 2 + lax.axis_index("c")
    base = wid * per_w

    def body(j, carry):
        off = base + j * CHUNK
        pltpu.sync_copy(idx_hbm.at[pl.ds(off, CHUNK)], idx_v)
        pltpu.async_copy(table_hbm.at[idx_v], rows_v, sem).wait()
        pltpu.sync_copy(rows_v, out_hbm.at[pl.ds(off, CHUNK)])
        return carry

    lax.fori_loop(0, n_chunk, body, 0)


def kernel(input_ids, weight):
    batch, seq = input_ids.shape
    n_flat = batch * seq
    per_w = n_flat // NUM_WORKERS
    n_chunk = per_w // CHUNK
    idx_flat = input_ids.reshape(n_flat).astype(jnp.int32)

    mesh = plsc.VectorSubcoreMesh(core_axis_name="c", subcore_axis_name="s")
    emb = functools.partial(
        pl.kernel,
        mesh=mesh,
        out_type=jax.ShapeDtypeStruct((n_flat, EMBED_DIM), jnp.float32),
        scratch_types=[
            pltpu.VMEM((CHUNK,), jnp.int32),
            pltpu.VMEM((CHUNK, EMBED_DIM), jnp.float32),
            pltpu.SemaphoreType.DMA,
        ],
        compiler_params=pltpu.CompilerParams(use_tc_tiling_on_sc=False),
    )(functools.partial(_emb_body, per_w=per_w, n_chunk=n_chunk))

    out = emb(idx_flat, weight)
    return out.reshape(batch, seq, EMBED_DIM)


# speedup vs baseline: 1.0473x; 1.0473x over previous
import functools

import jax
import jax.numpy as jnp
from jax import lax
from jax.experimental import pallas as pl
from jax.experimental.pallas import tpu as pltpu
from jax.experimental.pallas import tpu_sc as plsc

EMBED_DIM = 64
NUM_CORES = 2
NUM_SUBCORES = 16
NUM_WORKERS = NUM_CORES * NUM_SUBCORES
CHUNK = 640
NBUF = 2


def _emb_body(idx_hbm, table_hbm, out_hbm, idx_v, rows_v, sem_g, sem_o0, sem_o1,
              *, per_w, n_chunk):
    wid = lax.axis_index("s") * NUM_CORES + lax.axis_index("c")
    base = wid * per_w
    pltpu.sync_copy(idx_hbm.at[pl.ds(base, per_w)], idx_v)

    sems_o = (sem_o0, sem_o1)
    n_groups = n_chunk // NBUF

    def out_slot(j):
        return out_hbm.at[pl.ds(base + j * CHUNK, CHUNK)]

    def drain_out(b):
        pltpu.make_async_copy(out_slot(0), rows_v.at[b], sems_o[b]).wait()

    def start_gather(j, b):
        return pltpu.async_copy(
            table_hbm.at[idx_v.at[pl.ds(j * CHUNK, CHUNK)]],
            rows_v.at[b], sem_g)

    def group(g, carry):
        for b in range(NBUF):
            j = g * NBUF + b
            prev = (b - 1) % NBUF

            @pl.when(g > 0)
            def _():
                drain_out(b)

            h = start_gather(j, b)

            if b == 0:
                @pl.when(g > 0)
                def _():
                    pltpu.async_copy(rows_v.at[prev], out_slot(j - 1),
                                     sems_o[prev])
            else:
                pltpu.async_copy(rows_v.at[prev], out_slot(j - 1),
                                 sems_o[prev])
            h.wait()
        return carry

    lax.fori_loop(0, n_groups, group, 0)

    pltpu.async_copy(rows_v.at[NBUF - 1], out_slot(n_chunk - 1),
                     sems_o[NBUF - 1])
    for b in range(NBUF):
        drain_out(b)


def kernel(input_ids, weight):
    batch, seq = input_ids.shape
    n_flat = batch * seq
    per_w = n_flat // NUM_WORKERS
    n_chunk = per_w // CHUNK
    idx_flat = input_ids.reshape(n_flat).astype(jnp.int32)

    mesh = plsc.VectorSubcoreMesh(core_axis_name="c", subcore_axis_name="s")
    emb = functools.partial(
        pl.kernel,
        mesh=mesh,
        out_type=jax.ShapeDtypeStruct((n_flat, EMBED_DIM), jnp.float32),
        scratch_types=[
            pltpu.VMEM((per_w,), jnp.int32),
            pltpu.VMEM((NBUF, CHUNK, EMBED_DIM), jnp.float32),
            pltpu.SemaphoreType.DMA,
            pltpu.SemaphoreType.DMA,
            pltpu.SemaphoreType.DMA,
        ],
        compiler_params=pltpu.CompilerParams(use_tc_tiling_on_sc=False),
    )(functools.partial(_emb_body, per_w=per_w, n_chunk=n_chunk))

    out = emb(idx_flat, weight)
    return out.reshape(batch, seq, EMBED_DIM)
